# baseline (speedup 1.0000x reference)
import jax
import jax.numpy as jnp
from jax import lax
from jax.experimental import pallas as pl
from jax.experimental.pallas import tpu as pltpu
from jax.experimental.pallas import tpu_sc as plsc

N_NODES = 10000
N_EDGES = 160000
F = 128
NBASIS = 4
YW = NBASIS * F

CHUNK = 32
NCHUNKS = N_EDGES // CHUNK
NC, NS = 2, 16
NW = NC * NS
NPAD = 10240
ROWS_PER_TILE = NPAD // NS
SUPER = 32

CH_BASE = NCHUNKS // NW
CH_REM = NCHUNKS % NW

_MM_BLOCK = 1000


def _mm_body(x_ref, w_ref, y_ref):
    y_ref[...] = jnp.dot(x_ref[...], w_ref[...],
                         preferred_element_type=jnp.float32)


def _add_body(p_ref, o_ref):
    o_ref[...] = p_ref[0] + p_ref[1]


def _sc_body(y_hbm, pk_hbm, part_hbm, pks_v, rows_v, msg_v, acc_sh,
             gsem0, gsem1, psem):
    c = lax.axis_index("c")
    s = lax.axis_index("s")
    w = s * NC + c

    @pl.loop(0, CHUNK)
    def _zero_rows(r):
        for fb in range(F // 16):
            msg_v[r, pl.ds(fb * 16, 16)] = jnp.zeros((16,), jnp.float32)

    for t in range(ROWS_PER_TILE // CHUNK):
        pltpu.sync_copy(msg_v, acc_sh.at[pl.ds(s * ROWS_PER_TILE + t * CHUNK,
                                               CHUNK)])
    plsc.subcore_barrier()

    n_chunks = CH_BASE + jnp.where(w < CH_REM, 1, 0)
    base = w * CH_BASE + jnp.minimum(w, CH_REM)
    n_super = (n_chunks + SUPER - 1) // SUPER

    lane = lax.iota(jnp.int32, 16)
    gsems = (gsem0, gsem1)
    rows_bufs = (rows_v.at[0], rows_v.at[1])

    def issue_pk(sp, sb):
        pltpu.async_copy(pk_hbm.at[pl.ds(base + sp * SUPER, SUPER)],
                         pks_v.at[sb], psem)

    def wait_pk(sb):
        pltpu.make_async_copy(pk_hbm.at[pl.ds(0, SUPER)], pks_v.at[sb],
                              psem).wait()

    def issue_gather(sb, u, b):
        pltpu.async_copy(y_hbm.at[pks_v.at[sb, u, 3]], rows_bufs[b], gsems[b])

    def wait_gather(sb, u, b):
        pltpu.make_async_copy(y_hbm.at[pks_v.at[sb, u, 3]], rows_bufs[b],
                              gsems[b]).wait()

    def compute(sb, u, b):
        pk = pks_v.at[sb, u]
        rows = rows_bufs[b]
        for g in range(CHUNK // 16):
            pvec = lax.bitcast_convert_type(pk[0, pl.ds(g * 16, 16)],
                                            jnp.float32)
            qvec = lax.bitcast_convert_type(pk[1, pl.ds(g * 16, 16)],
                                            jnp.float32)
            one = jnp.float32(1.0)
            half = jnp.float32(0.5)
            zero = jnp.float32(0.0)
            u0 = jnp.maximum(zero, one - half * jnp.abs(pvec + one))
            u1 = jnp.maximum(zero, one - half * jnp.abs(pvec - one))
            v0 = jnp.maximum(zero, one - half * jnp.abs(qvec + one))
            v1 = jnp.maximum(zero, one - half * jnp.abs(qvec - one))
            c00 = u0 * v0
            c01 = u0 * v1
            c10 = u1 * v0
            c11 = u1 * v1
            eidx = lane + g * 16

            @pl.loop(0, F, unroll=4)
            def _f(f):
                fcol = jnp.full((16,), f, jnp.int32)
                a0 = plsc.load_gather(rows, [eidx, fcol])
                a1 = plsc.load_gather(rows, [eidx, fcol + F])
                a2 = plsc.load_gather(rows, [eidx, fcol + 2 * F])
                a3 = plsc.load_gather(rows, [eidx, fcol + 3 * F])
                acc = a0 * c00 + a1 * c01 + a2 * c10 + a3 * c11
                plsc.store_scatter(msg_v, [eidx, fcol], acc)

        pltpu.sync_copy(msg_v, acc_sh.at[pks_v.at[sb, u, 2]], add=True)

    issue_pk(0, 0)
    wait_pk(0)

    @pl.when(n_super > 1)
    def _():
        issue_pk(1, 1)

    @pl.loop(0, n_super)
    def _super(sp):
        sb = sp & 1
        nsc = jnp.minimum(n_chunks - sp * SUPER, SUPER)
        issue_gather(sb, 0, 0)

        @pl.loop(0, (nsc + 1) // 2)
        def _pair(up):
            u0 = 2 * up

            @pl.when(u0 + 1 < nsc)
            def _():
                issue_gather(sb, u0 + 1, 1)

            wait_gather(sb, u0, 0)
            compute(sb, u0, 0)

            @pl.when(u0 + 2 < nsc)
            def _():
                issue_gather(sb, u0 + 2, 0)

            @pl.when(u0 + 1 < nsc)
            def _():
                wait_gather(sb, u0 + 1, 1)
                compute(sb, u0 + 1, 1)

        @pl.when(sp + 1 < n_super)
        def _():
            wait_pk(1 - sb)

        @pl.when(sp + 2 < n_super)
        def _():
            issue_pk(sp + 2, sb)

    plsc.subcore_barrier()
    pltpu.sync_copy(acc_sh.at[pl.ds(s * ROWS_PER_TILE, ROWS_PER_TILE)],
                    part_hbm.at[c, pl.ds(s * ROWS_PER_TILE, ROWS_PER_TILE)])


_sc_call = pl.kernel(
    _sc_body,
    out_type=jax.ShapeDtypeStruct((NC, NPAD, F), jnp.float32),
    mesh=plsc.VectorSubcoreMesh(core_axis_name="c", subcore_axis_name="s"),
    compiler_params=pltpu.CompilerParams(use_tc_tiling_on_sc=False,
                                         needs_layout_passes=False),
    scratch_types=[
        pltpu.VMEM((2, SUPER, 4, CHUNK), jnp.int32),
        pltpu.VMEM((2, CHUNK, YW), jnp.float32),
        pltpu.VMEM((CHUNK, F), jnp.float32),
        pltpu.VMEM_SHARED((NPAD, F), jnp.float32),
        pltpu.SemaphoreType.DMA,
        pltpu.SemaphoreType.DMA,
        pltpu.SemaphoreType.DMA,
    ],
)


@jax.jit
def kernel(x, edge_index, edge_attr, weight):
    w_flat = weight.transpose(2, 0, 1, 3).reshape(F, YW)
    grid = N_NODES // _MM_BLOCK
    y = pl.pallas_call(
        _mm_body,
        grid=(grid,),
        in_specs=[
            pl.BlockSpec((_MM_BLOCK, F), lambda i: (i, 0)),
            pl.BlockSpec((F, YW), lambda i: (0, 0)),
        ],
        out_specs=pl.BlockSpec((_MM_BLOCK, YW), lambda i: (i, 0)),
        out_shape=jax.ShapeDtypeStruct((N_NODES, YW), jnp.float32),
    )(x, w_flat)

    pb = lax.bitcast_convert_type(edge_attr[:, 0], jnp.int32)
    qb = lax.bitcast_convert_type(edge_attr[:, 1], jnp.int32)
    packed = (jnp.stack([pb, qb, edge_index[0], edge_index[1]], axis=0)
              .reshape(4, NCHUNKS, CHUNK).transpose(1, 0, 2))

    partials = _sc_call(y, packed)

    out = pl.pallas_call(
        _add_body,
        grid=(grid,),
        in_specs=[pl.BlockSpec((NC, _MM_BLOCK, F), lambda i: (0, i, 0))],
        out_specs=pl.BlockSpec((_MM_BLOCK, F), lambda i: (i, 0)),
        out_shape=jax.ShapeDtypeStruct((N_NODES, F), jnp.float32),
    )(partials)
    return out


# speedup vs baseline: 4.1553x; 4.1553x over previous
import jax
import jax.numpy as jnp
from jax import lax
from jax.experimental import pallas as pl
from jax.experimental.pallas import tpu as pltpu
from jax.experimental.pallas import tpu_sc as plsc

N_NODES = 10000
N_EDGES = 160000
F = 128
NBASIS = 4
YW = NBASIS * F

CHUNK = 32
NCHUNKS = N_EDGES // CHUNK
NC, NS = 2, 16
NW = NC * NS
NPAD = 10240
ROWS_PER_TILE = NPAD // NS
SUPER = 32

CH_BASE = NCHUNKS // NW
CH_REM = NCHUNKS % NW

_MM_BLOCK = 1000


def _mm_body(x_ref, w_ref, y_ref):
    y_ref[...] = jnp.dot(x_ref[...], w_ref[...],
                         preferred_element_type=jnp.float32)


def _add_body(p_ref, o_ref):
    o_ref[...] = p_ref[0] + p_ref[1]


def _sc_body(y_hbm, pk_hbm, part_hbm, pks_v, rows_v, msg_v, acc_sh,
             gsem0, gsem1, psem):
    c = lax.axis_index("c")
    s = lax.axis_index("s")
    w = s * NC + c

    @pl.loop(0, CHUNK)
    def _zero_rows(r):
        for fb in range(F // 16):
            msg_v[r, pl.ds(fb * 16, 16)] = jnp.zeros((16,), jnp.float32)

    for t in range(ROWS_PER_TILE // CHUNK):
        pltpu.sync_copy(msg_v, acc_sh.at[pl.ds(s * ROWS_PER_TILE + t * CHUNK,
                                               CHUNK)])
    plsc.subcore_barrier()

    n_chunks = CH_BASE + jnp.where(w < CH_REM, 1, 0)
    base = w * CH_BASE + jnp.minimum(w, CH_REM)
    n_super = (n_chunks + SUPER - 1) // SUPER

    gsems = (gsem0, gsem1)
    rows_bufs = (rows_v.at[0], rows_v.at[1])

    def issue_pk(sp, sb):
        pltpu.async_copy(pk_hbm.at[pl.ds(base + sp * SUPER, SUPER)],
                         pks_v.at[sb], psem)

    def wait_pk(sb):
        pltpu.make_async_copy(pk_hbm.at[pl.ds(0, SUPER)], pks_v.at[sb],
                              psem).wait()

    def issue_gather(sb, u, b):
        pltpu.async_copy(y_hbm.at[pks_v.at[sb, u, 3]], rows_bufs[b], gsems[b])

    def wait_gather(sb, u, b):
        pltpu.make_async_copy(y_hbm.at[pks_v.at[sb, u, 3]], rows_bufs[b],
                              gsems[b]).wait()

    def compute(sb, u, b):
        pk = pks_v.at[sb, u]
        rows = rows_bufs[b]
        for g in range(CHUNK // 16):
            pvec = lax.bitcast_convert_type(pk[0, pl.ds(g * 16, 16)],
                                            jnp.float32)
            qvec = lax.bitcast_convert_type(pk[1, pl.ds(g * 16, 16)],
                                            jnp.float32)
            one = jnp.float32(1.0)
            half = jnp.float32(0.5)
            zero = jnp.float32(0.0)
            u0 = jnp.maximum(zero, one - half * jnp.abs(pvec + one))
            u1 = jnp.maximum(zero, one - half * jnp.abs(pvec - one))
            v0 = jnp.maximum(zero, one - half * jnp.abs(qvec + one))
            v1 = jnp.maximum(zero, one - half * jnp.abs(qvec - one))
            c00 = u0 * v0
            c01 = u0 * v1
            c10 = u1 * v0
            c11 = u1 * v1

            @pl.loop(0, 16)
            def _edge(k):
                kvec = jnp.full((16,), k, jnp.int32)
                b0 = c00[kvec]
                b1 = c01[kvec]
                b2 = c10[kvec]
                b3 = c11[kvec]
                e = g * 16 + k
                for fb in range(F // 16):
                    o = fb * 16
                    acc = rows[e, pl.ds(o, 16)] * b0
                    acc = acc + rows[e, pl.ds(F + o, 16)] * b1
                    acc = acc + rows[e, pl.ds(2 * F + o, 16)] * b2
                    acc = acc + rows[e, pl.ds(3 * F + o, 16)] * b3
                    msg_v[e, pl.ds(o, 16)] = acc

        pltpu.sync_copy(msg_v, acc_sh.at[pks_v.at[sb, u, 2]], add=True)

    issue_pk(0, 0)
    wait_pk(0)

    @pl.when(n_super > 1)
    def _():
        issue_pk(1, 1)

    @pl.loop(0, n_super)
    def _super(sp):
        sb = sp & 1
        nsc = jnp.minimum(n_chunks - sp * SUPER, SUPER)
        issue_gather(sb, 0, 0)

        @pl.loop(0, (nsc + 1) // 2)
        def _pair(up):
            u0 = 2 * up

            @pl.when(u0 + 1 < nsc)
            def _():
                issue_gather(sb, u0 + 1, 1)

            wait_gather(sb, u0, 0)
            compute(sb, u0, 0)

            @pl.when(u0 + 2 < nsc)
            def _():
                issue_gather(sb, u0 + 2, 0)

            @pl.when(u0 + 1 < nsc)
            def _():
                wait_gather(sb, u0 + 1, 1)
                compute(sb, u0 + 1, 1)

        @pl.when(sp + 1 < n_super)
        def _():
            wait_pk(1 - sb)

        @pl.when(sp + 2 < n_super)
        def _():
            issue_pk(sp + 2, sb)

    plsc.subcore_barrier()
    pltpu.sync_copy(acc_sh.at[pl.ds(s * ROWS_PER_TILE, ROWS_PER_TILE)],
                    part_hbm.at[c, pl.ds(s * ROWS_PER_TILE, ROWS_PER_TILE)])


_sc_call = pl.kernel(
    _sc_body,
    out_type=jax.ShapeDtypeStruct((NC, NPAD, F), jnp.float32),
    mesh=plsc.VectorSubcoreMesh(core_axis_name="c", subcore_axis_name="s"),
    compiler_params=pltpu.CompilerParams(use_tc_tiling_on_sc=False,
                                         needs_layout_passes=False),
    scratch_types=[
        pltpu.VMEM((2, SUPER, 4, CHUNK), jnp.int32),
        pltpu.VMEM((2, CHUNK, YW), jnp.float32),
        pltpu.VMEM((CHUNK, F), jnp.float32),
        pltpu.VMEM_SHARED((NPAD, F), jnp.float32),
        pltpu.SemaphoreType.DMA,
        pltpu.SemaphoreType.DMA,
        pltpu.SemaphoreType.DMA,
    ],
)


@jax.jit
def kernel(x, edge_index, edge_attr, weight):
    w_flat = weight.transpose(2, 0, 1, 3).reshape(F, YW)
    grid = N_NODES // _MM_BLOCK
    y = pl.pallas_call(
        _mm_body,
        grid=(grid,),
        in_specs=[
            pl.BlockSpec((_MM_BLOCK, F), lambda i: (i, 0)),
            pl.BlockSpec((F, YW), lambda i: (0, 0)),
        ],
        out_specs=pl.BlockSpec((_MM_BLOCK, YW), lambda i: (i, 0)),
        out_shape=jax.ShapeDtypeStruct((N_NODES, YW), jnp.float32),
    )(x, w_flat)

    pb = lax.bitcast_convert_type(edge_attr[:, 0], jnp.int32)
    qb = lax.bitcast_convert_type(edge_attr[:, 1], jnp.int32)
    packed = (jnp.stack([pb, qb, edge_index[0], edge_index[1]], axis=0)
              .reshape(4, NCHUNKS, CHUNK).transpose(1, 0, 2))

    partials = _sc_call(y, packed)

    out = pl.pallas_call(
        _add_body,
        grid=(grid,),
        in_specs=[pl.BlockSpec((NC, _MM_BLOCK, F), lambda i: (0, i, 0))],
        out_specs=pl.BlockSpec((_MM_BLOCK, F), lambda i: (i, 0)),
        out_shape=jax.ShapeDtypeStruct((N_NODES, F), jnp.float32),
    )(partials)
    return out


# speedup vs baseline: 5.5396x; 1.3331x over previous
import jax
import jax.numpy as jnp
import numpy as np
from jax import lax
from jax.experimental import pallas as pl
from jax.experimental.pallas import tpu as pltpu
from jax.experimental.pallas import tpu_sc as plsc

N_NODES = 10000
N_EDGES = 160000
F = 128
NBASIS = 4
YW = NBASIS * F

CHUNK = 32
NCHUNKS = N_EDGES // CHUNK
NC, NS = 2, 16
NW = NC * NS
NPAD = 10240
ROWS_PER_TILE = NPAD // NS
SUPER = 32

CH_BASE = NCHUNKS // NW
CH_REM = NCHUNKS % NW

_MM_BLOCK = 2000

_PERM = np.empty(YW, np.int32)
for _g in range(YW // 32):
    for _t in range(16):
        _PERM[_g * 32 + 2 * _t] = _g * 32 + _t
        _PERM[_g * 32 + 2 * _t + 1] = _g * 32 + 16 + _t


def _mm_body(x_ref, w_ref, y_ref):
    y_ref[...] = jnp.dot(x_ref[...], w_ref[...],
                         preferred_element_type=jnp.float32
                         ).astype(jnp.bfloat16)


def _add_body(p_ref, o_ref):
    o_ref[...] = p_ref[0] + p_ref[1]


def _sc_body(y_hbm, pk_hbm, part_hbm, pks_v, rows_v, msg_v, acc_sh,
             gsem0, gsem1, psem):
    c = lax.axis_index("c")
    s = lax.axis_index("s")
    w = s * NC + c

    @pl.loop(0, CHUNK)
    def _zero_rows(r):
        for fb in range(F // 16):
            msg_v[r, pl.ds(fb * 16, 16)] = jnp.zeros((16,), jnp.float32)

    for t in range(ROWS_PER_TILE // CHUNK):
        pltpu.sync_copy(msg_v, acc_sh.at[pl.ds(s * ROWS_PER_TILE + t * CHUNK,
                                               CHUNK)])
    plsc.subcore_barrier()

    n_chunks = CH_BASE + jnp.where(w < CH_REM, 1, 0)
    base = w * CH_BASE + jnp.minimum(w, CH_REM)
    n_super = (n_chunks + SUPER - 1) // SUPER

    gsems = (gsem0, gsem1)
    rows_bufs = (rows_v.at[0], rows_v.at[1])

    def issue_pk(sp, sb):
        pltpu.async_copy(pk_hbm.at[pl.ds(base + sp * SUPER, SUPER)],
                         pks_v.at[sb], psem)

    def wait_pk(sb):
        pltpu.make_async_copy(pk_hbm.at[pl.ds(0, SUPER)], pks_v.at[sb],
                              psem).wait()

    def issue_gather(sb, u, b):
        pltpu.async_copy(y_hbm.at[pks_v.at[sb, u, 3]], rows_bufs[b], gsems[b])

    def wait_gather(sb, u, b):
        pltpu.make_async_copy(y_hbm.at[pks_v.at[sb, u, 3]], rows_bufs[b],
                              gsems[b]).wait()

    def compute(sb, u, b):
        pk = pks_v.at[sb, u]
        rows = rows_bufs[b]
        for g in range(CHUNK // 16):
            pvec = lax.bitcast_convert_type(pk[0, pl.ds(g * 16, 16)],
                                            jnp.float32)
            qvec = lax.bitcast_convert_type(pk[1, pl.ds(g * 16, 16)],
                                            jnp.float32)
            one = jnp.float32(1.0)
            half = jnp.float32(0.5)
            zero = jnp.float32(0.0)
            u0 = jnp.maximum(zero, one - half * jnp.abs(pvec + one))
            u1 = jnp.maximum(zero, one - half * jnp.abs(pvec - one))
            v0 = jnp.maximum(zero, one - half * jnp.abs(qvec + one))
            v1 = jnp.maximum(zero, one - half * jnp.abs(qvec - one))
            c00 = u0 * v0
            c01 = u0 * v1
            c10 = u1 * v0
            c11 = u1 * v1

            @pl.loop(0, 16)
            def _edge(k):
                kvec = jnp.full((16,), k, jnp.int32)
                b0 = c00[kvec]
                b1 = c01[kvec]
                b2 = c10[kvec]
                b3 = c11[kvec]
                bc = (b0, b1, b2, b3)
                e = g * 16 + k
                for fb in range(F // 32):
                    o = fb * 32
                    acc_a = jnp.zeros((16,), jnp.float32)
                    acc_b = jnp.zeros((16,), jnp.float32)
                    for ab in range(NBASIS):
                        raw = rows[e, pl.ds(ab * F + o, 32)]
                        ra, rb = plsc.unpack(raw,
                                             format=plsc.PackFormat.INTERLEAVED)
                        acc_a = acc_a + ra * bc[ab]
                        acc_b = acc_b + rb * bc[ab]
                    msg_v[e, pl.ds(o, 16)] = acc_a
                    msg_v[e, pl.ds(o + 16, 16)] = acc_b

        pltpu.sync_copy(msg_v, acc_sh.at[pks_v.at[sb, u, 2]], add=True)

    issue_pk(0, 0)
    wait_pk(0)

    @pl.when(n_super > 1)
    def _():
        issue_pk(1, 1)

    @pl.loop(0, n_super)
    def _super(sp):
        sb = sp & 1
        nsc = jnp.minimum(n_chunks - sp * SUPER, SUPER)
        issue_gather(sb, 0, 0)

        @pl.loop(0, (nsc + 1) // 2)
        def _pair(up):
            u0 = 2 * up

            @pl.when(u0 + 1 < nsc)
            def _():
                issue_gather(sb, u0 + 1, 1)

            wait_gather(sb, u0, 0)
            compute(sb, u0, 0)

            @pl.when(u0 + 2 < nsc)
            def _():
                issue_gather(sb, u0 + 2, 0)

            @pl.when(u0 + 1 < nsc)
            def _():
                wait_gather(sb, u0 + 1, 1)
                compute(sb, u0 + 1, 1)

        @pl.when(sp + 1 < n_super)
        def _():
            wait_pk(1 - sb)

        @pl.when(sp + 2 < n_super)
        def _():
            issue_pk(sp + 2, sb)

    plsc.subcore_barrier()
    pltpu.sync_copy(acc_sh.at[pl.ds(s * ROWS_PER_TILE, ROWS_PER_TILE)],
                    part_hbm.at[c, pl.ds(s * ROWS_PER_TILE, ROWS_PER_TILE)])


_sc_call = pl.kernel(
    _sc_body,
    out_type=jax.ShapeDtypeStruct((NC, NPAD, F), jnp.float32),
    mesh=plsc.VectorSubcoreMesh(core_axis_name="c", subcore_axis_name="s"),
    compiler_params=pltpu.CompilerParams(use_tc_tiling_on_sc=False,
                                         needs_layout_passes=False),
    scratch_types=[
        pltpu.VMEM((2, SUPER, 4, CHUNK), jnp.int32),
        pltpu.VMEM((2, CHUNK, YW), jnp.bfloat16),
        pltpu.VMEM((CHUNK, F), jnp.float32),
        pltpu.VMEM_SHARED((NPAD, F), jnp.float32),
        pltpu.SemaphoreType.DMA,
        pltpu.SemaphoreType.DMA,
        pltpu.SemaphoreType.DMA,
    ],
)


@jax.jit
def kernel(x, edge_index, edge_attr, weight):
    w_flat = weight.transpose(2, 0, 1, 3).reshape(F, YW)[:, _PERM]
    grid = N_NODES // _MM_BLOCK
    y = pl.pallas_call(
        _mm_body,
        grid=(grid,),
        in_specs=[
            pl.BlockSpec((_MM_BLOCK, F), lambda i: (i, 0)),
            pl.BlockSpec((F, YW), lambda i: (0, 0)),
        ],
        out_specs=pl.BlockSpec((_MM_BLOCK, YW), lambda i: (i, 0)),
        out_shape=jax.ShapeDtypeStruct((N_NODES, YW), jnp.bfloat16),
    )(x, w_flat)

    pb = lax.bitcast_convert_type(edge_attr[:, 0], jnp.int32)
    qb = lax.bitcast_convert_type(edge_attr[:, 1], jnp.int32)
    packed = (jnp.stack([pb, qb, edge_index[0], edge_index[1]], axis=0)
              .reshape(4, NCHUNKS, CHUNK).transpose(1, 0, 2))

    partials = _sc_call(y, packed)

    out = pl.pallas_call(
        _add_body,
        grid=(grid,),
        in_specs=[pl.BlockSpec((NC, _MM_BLOCK, F), lambda i: (0, i, 0))],
        out_specs=pl.BlockSpec((_MM_BLOCK, F), lambda i: (i, 0)),
        out_shape=jax.ShapeDtypeStruct((N_NODES, F), jnp.float32),
    )(partials)
    return out


# speedup vs baseline: 5.9963x; 1.0825x over previous
import jax
import jax.numpy as jnp
import numpy as np
from jax import lax
from jax.experimental import pallas as pl
from jax.experimental.pallas import tpu as pltpu
from jax.experimental.pallas import tpu_sc as plsc

N_NODES = 10000
N_EDGES = 160000
F = 128
NBASIS = 4
YW = NBASIS * F

CHUNK = 32
NCHUNKS = N_EDGES // CHUNK
NC, NS = 2, 16
NW = NC * NS
NPAD = 10240
ROWS_PER_TILE = NPAD // NS
SUPER = 32

CH_BASE = NCHUNKS // NW
CH_REM = NCHUNKS % NW

_MM_BLOCK = 2000

_PERM = np.empty(YW, np.int32)
for _g in range(YW // 32):
    for _t in range(16):
        _PERM[_g * 32 + 2 * _t] = _g * 32 + _t
        _PERM[_g * 32 + 2 * _t + 1] = _g * 32 + 16 + _t


def _mm_body(x_ref, w_ref, y_ref):
    y_ref[...] = jnp.dot(x_ref[...], w_ref[...],
                         preferred_element_type=jnp.float32
                         ).astype(jnp.bfloat16)


def _add_body(p_ref, o_ref):
    o_ref[...] = p_ref[0] + p_ref[1]


def _sc_body(y_hbm, pk_hbm, part_hbm, pks_v, rows_v, msg_v, acc_sh,
             gsem0, gsem1, gsem2, gsem3, psem, ssem0, ssem1):
    c = lax.axis_index("c")
    s = lax.axis_index("s")
    w = s * NC + c

    for mh in range(2):
        @pl.loop(0, CHUNK)
        def _zero_rows(r):
            for fb in range(F // 16):
                msg_v[mh, r, pl.ds(fb * 16, 16)] = jnp.zeros((16,),
                                                             jnp.float32)

    for t in range(ROWS_PER_TILE // CHUNK):
        pltpu.sync_copy(msg_v.at[t % 2],
                        acc_sh.at[pl.ds(s * ROWS_PER_TILE + t * CHUNK,
                                        CHUNK)])
    plsc.subcore_barrier()

    n_chunks = CH_BASE + jnp.where(w < CH_REM, 1, 0)
    base = w * CH_BASE + jnp.minimum(w, CH_REM)
    n_super = (n_chunks + SUPER - 1) // SUPER

    gsems = (gsem0, gsem1, gsem2, gsem3)
    ssems = (ssem0, ssem1)
    rows_bufs = (rows_v.at[0], rows_v.at[1], rows_v.at[2], rows_v.at[3])

    def issue_pk(sp, sb):
        pltpu.async_copy(pk_hbm.at[pl.ds(base + sp * SUPER, SUPER)],
                         pks_v.at[sb], psem)

    def wait_pk(sb):
        pltpu.make_async_copy(pk_hbm.at[pl.ds(0, SUPER)], pks_v.at[sb],
                              psem).wait()

    def issue_gather(sb, u, b):
        pltpu.async_copy(y_hbm.at[pks_v.at[sb, u, 2]], rows_bufs[b], gsems[b])

    def wait_gather(sb, u, b):
        pltpu.make_async_copy(y_hbm.at[pks_v.at[sb, u, 2]], rows_bufs[b],
                              gsems[b]).wait()

    def wait_scatter(mh):
        pltpu.make_async_copy(msg_v.at[mh], acc_sh.at[pks_v.at[0, 0, 1]],
                              ssems[mh]).wait()

    def compute(sb, u, b, mh):
        pk = pks_v.at[sb, u]
        rows = rows_bufs[b]
        wait_scatter(mh)
        for g in range(CHUNK // 16):
            pq = plsc.bitcast(pk[0, pl.ds(g * 16, 16)], jnp.bfloat16)
            pvec, qvec = plsc.unpack(pq, format=plsc.PackFormat.INTERLEAVED)
            one = jnp.float32(1.0)
            half = jnp.float32(0.5)
            zero = jnp.float32(0.0)
            u0 = jnp.maximum(zero, one - half * jnp.abs(pvec + one))
            u1 = jnp.maximum(zero, one - half * jnp.abs(pvec - one))
            v0 = jnp.maximum(zero, one - half * jnp.abs(qvec + one))
            v1 = jnp.maximum(zero, one - half * jnp.abs(qvec - one))
            c00 = u0 * v0
            c01 = u0 * v1
            c10 = u1 * v0
            c11 = u1 * v1

            @pl.loop(0, 16)
            def _edge(k):
                kvec = jnp.full((16,), k, jnp.int32)
                b0 = c00[kvec]
                b1 = c01[kvec]
                b2 = c10[kvec]
                b3 = c11[kvec]
                bc = (b0, b1, b2, b3)
                e = g * 16 + k
                for fb in range(F // 32):
                    o = fb * 32
                    acc_a = jnp.zeros((16,), jnp.float32)
                    acc_b = jnp.zeros((16,), jnp.float32)
                    for ab in range(NBASIS):
                        raw = rows[e, pl.ds(ab * F + o, 32)]
                        ra, rb = plsc.unpack(raw,
                                             format=plsc.PackFormat.INTERLEAVED)
                        acc_a = acc_a + ra * bc[ab]
                        acc_b = acc_b + rb * bc[ab]
                    msg_v[mh, e, pl.ds(o, 16)] = acc_a
                    msg_v[mh, e, pl.ds(o + 16, 16)] = acc_b

        pltpu.async_copy(msg_v.at[mh], acc_sh.at[pks_v.at[sb, u, 1]],
                         ssems[mh], add=True)

    issue_pk(0, 0)
    wait_pk(0)

    for mh in range(2):
        pltpu.async_copy(msg_v.at[mh], acc_sh.at[pks_v.at[0, 0, 1]],
                         ssems[mh], add=True)

    @pl.loop(0, n_super)
    def _super(sp):
        sb = sp & 1
        nsc = jnp.minimum(n_chunks - sp * SUPER, SUPER)
        issue_gather(sb, 0, 0)

        @pl.when(1 < nsc)
        def _():
            issue_gather(sb, 1, 1)

        @pl.when(2 < nsc)
        def _():
            issue_gather(sb, 2, 2)

        @pl.loop(0, (nsc + 3) // 4)
        def _quad(tp):
            u0 = 4 * tp
            for j in range(4):
                u = u0 + j

                @pl.when(u < nsc)
                def _():
                    @pl.when(u + 3 < nsc)
                    def _():
                        issue_gather(sb, u + 3, (j + 3) % 4)

                    wait_gather(sb, u, j)
                    compute(sb, u, j, j % 2)

            @pl.when((tp == 0) & ((sp + 1) * SUPER < n_chunks))
            def _():
                issue_pk(sp + 1, 1 - sb)

        @pl.when((sp + 1) * SUPER < n_chunks)
        def _():
            wait_pk(1 - sb)

    wait_scatter(0)
    wait_scatter(1)
    plsc.subcore_barrier()
    pltpu.sync_copy(acc_sh.at[pl.ds(s * ROWS_PER_TILE, ROWS_PER_TILE)],
                    part_hbm.at[c, pl.ds(s * ROWS_PER_TILE, ROWS_PER_TILE)])


_sc_call = pl.kernel(
    _sc_body,
    out_type=jax.ShapeDtypeStruct((NC, NPAD, F), jnp.float32),
    mesh=plsc.VectorSubcoreMesh(core_axis_name="c", subcore_axis_name="s"),
    compiler_params=pltpu.CompilerParams(use_tc_tiling_on_sc=False,
                                         needs_layout_passes=False),
    scratch_types=[
        pltpu.VMEM((2, SUPER, 3, CHUNK), jnp.int32),
        pltpu.VMEM((4, CHUNK, YW), jnp.bfloat16),
        pltpu.VMEM((2, CHUNK, F), jnp.float32),
        pltpu.VMEM_SHARED((NPAD, F), jnp.float32),
        pltpu.SemaphoreType.DMA,
        pltpu.SemaphoreType.DMA,
        pltpu.SemaphoreType.DMA,
        pltpu.SemaphoreType.DMA,
        pltpu.SemaphoreType.DMA,
        pltpu.SemaphoreType.DMA,
        pltpu.SemaphoreType.DMA,
    ],
)


@jax.jit
def kernel(x, edge_index, edge_attr, weight):
    w_flat = weight.transpose(2, 0, 1, 3).reshape(F, YW)[:, _PERM]
    grid = N_NODES // _MM_BLOCK
    y = pl.pallas_call(
        _mm_body,
        grid=(grid,),
        in_specs=[
            pl.BlockSpec((_MM_BLOCK, F), lambda i: (i, 0)),
            pl.BlockSpec((F, YW), lambda i: (0, 0)),
        ],
        out_specs=pl.BlockSpec((_MM_BLOCK, YW), lambda i: (i, 0)),
        out_shape=jax.ShapeDtypeStruct((N_NODES, YW), jnp.bfloat16),
    )(x, w_flat)

    pq = lax.bitcast_convert_type(
        jnp.stack([edge_attr[:, 0].astype(jnp.bfloat16),
                   edge_attr[:, 1].astype(jnp.bfloat16)], axis=-1),
        jnp.int32)
    packed = (jnp.stack([pq, edge_index[0], edge_index[1]], axis=0)
              .reshape(3, NCHUNKS, CHUNK).transpose(1, 0, 2))

    partials = _sc_call(y, packed)

    out = pl.pallas_call(
        _add_body,
        grid=(grid,),
        in_specs=[pl.BlockSpec((NC, _MM_BLOCK, F), lambda i: (0, i, 0))],
        out_specs=pl.BlockSpec((_MM_BLOCK, F), lambda i: (i, 0)),
        out_shape=jax.ShapeDtypeStruct((N_NODES, F), jnp.float32),
    )(partials)
    return out


# speedup vs baseline: 6.5975x; 1.1003x over previous
import jax
import jax.numpy as jnp
import numpy as np
from jax import lax
from jax.experimental import pallas as pl
from jax.experimental.pallas import tpu as pltpu
from jax.experimental.pallas import tpu_sc as plsc

N_NODES = 10000
N_EDGES = 160000
F = 128
NBASIS = 4
YW = NBASIS * F

SCH = 32
GCH = 64
NGCH = N_EDGES // GCH
NSCH = N_EDGES // SCH
NC, NS = 2, 16
NW = NC * NS
NPAD = 10240
ROWS_PER_TILE = NPAD // NS
SUPER = 16

G_BASE = NGCH // NW
G_REM = NGCH % NW

_MM_BLOCK = 2000

_PERM = np.empty(YW, np.int32)
for _g in range(YW // 32):
    for _t in range(16):
        _PERM[_g * 32 + 2 * _t] = _g * 32 + _t
        _PERM[_g * 32 + 2 * _t + 1] = _g * 32 + 16 + _t


def _mm_body(x_ref, w_ref, y_ref):
    y_ref[...] = jnp.dot(x_ref[...], w_ref[...],
                         preferred_element_type=jnp.float32
                         ).astype(jnp.bfloat16)


def _add_body(p_ref, o_ref):
    o_ref[...] = p_ref[0] + p_ref[1]


def _sc_body(y_hbm, jg_hbm, i2_hbm, pq_hbm, part_hbm,
             js_v, is_v, pqs_v, rows_v, msg_v, acc_sh,
             gsem0, gsem1, psem, ssem0, ssem1):
    c = lax.axis_index("c")
    s = lax.axis_index("s")
    w = s * NC + c

    for mh in range(2):
        @pl.loop(0, SCH)
        def _zero_rows(r):
            for fb in range(F // 16):
                msg_v[mh, r, pl.ds(fb * 16, 16)] = jnp.zeros((16,),
                                                             jnp.float32)

    for t in range(ROWS_PER_TILE // SCH):
        pltpu.sync_copy(msg_v.at[t % 2],
                        acc_sh.at[pl.ds(s * ROWS_PER_TILE + t * SCH, SCH)])
    plsc.subcore_barrier()

    n_gch = G_BASE + jnp.where(w < G_REM, 1, 0)
    base_g = w * G_BASE + jnp.minimum(w, G_REM)

    gsems = (gsem0, gsem1)
    ssems = (ssem0, ssem1)
    rows_bufs = (rows_v.at[0], rows_v.at[1])
    n_super = (n_gch + SUPER - 1) // SUPER

    def issue_pk(sp, sb):
        pltpu.async_copy(jg_hbm.at[pl.ds(base_g + sp * SUPER, SUPER)],
                         js_v.at[sb], psem)
        sbase = 2 * (base_g + sp * SUPER)
        pltpu.async_copy(i2_hbm.at[pl.ds(sbase, 2 * SUPER)], is_v.at[sb],
                         psem)
        pltpu.async_copy(pq_hbm.at[pl.ds(sbase, 2 * SUPER)], pqs_v.at[sb],
                         psem)

    def wait_pk(sb):
        pltpu.make_async_copy(jg_hbm.at[pl.ds(0, SUPER)], js_v.at[sb],
                              psem).wait()
        pltpu.make_async_copy(i2_hbm.at[pl.ds(0, 2 * SUPER)], is_v.at[sb],
                              psem).wait()
        pltpu.make_async_copy(pq_hbm.at[pl.ds(0, 2 * SUPER)], pqs_v.at[sb],
                              psem).wait()

    def issue_gather(sb, u, b):
        pltpu.async_copy(y_hbm.at[js_v.at[sb, u]], rows_bufs[b], gsems[b])

    def wait_gather(sb, u, b):
        pltpu.make_async_copy(y_hbm.at[js_v.at[sb, u]], rows_bufs[b],
                              gsems[b]).wait()

    def wait_scatter(mh):
        pltpu.make_async_copy(msg_v.at[mh], acc_sh.at[is_v.at[0, 0]],
                              ssems[mh]).wait()

    def compute_sub(sb, su, rows, off, mh):
        wait_scatter(mh)
        for g in range(SCH // 16):
            pq = plsc.bitcast(pqs_v[sb, su, pl.ds(g * 16, 16)], jnp.bfloat16)
            pvec, qvec = plsc.unpack(pq, format=plsc.PackFormat.INTERLEAVED)
            one = jnp.float32(1.0)
            half = jnp.float32(0.5)
            zero = jnp.float32(0.0)
            u0 = jnp.maximum(zero, one - half * jnp.abs(pvec + one))
            u1 = jnp.maximum(zero, one - half * jnp.abs(pvec - one))
            v0 = jnp.maximum(zero, one - half * jnp.abs(qvec + one))
            v1 = jnp.maximum(zero, one - half * jnp.abs(qvec - one))
            c00 = u0 * v0
            c01 = u0 * v1
            c10 = u1 * v0
            c11 = u1 * v1

            @pl.loop(0, 16)
            def _edge(k):
                kvec = jnp.full((16,), k, jnp.int32)
                b0 = c00[kvec]
                b1 = c01[kvec]
                b2 = c10[kvec]
                b3 = c11[kvec]
                bc = (b0, b1, b2, b3)
                e = g * 16 + k
                for fb in range(F // 32):
                    o = fb * 32
                    acc_a = jnp.zeros((16,), jnp.float32)
                    acc_b = jnp.zeros((16,), jnp.float32)
                    for ab in range(NBASIS):
                        raw = rows[off + e, pl.ds(ab * F + o, 32)]
                        ra, rb = plsc.unpack(raw,
                                             format=plsc.PackFormat.INTERLEAVED)
                        acc_a = acc_a + ra * bc[ab]
                        acc_b = acc_b + rb * bc[ab]
                    msg_v[mh, e, pl.ds(o, 16)] = acc_a
                    msg_v[mh, e, pl.ds(o + 16, 16)] = acc_b

        pltpu.async_copy(msg_v.at[mh], acc_sh.at[is_v.at[sb, su]],
                         ssems[mh], add=True)

    issue_pk(0, 0)
    wait_pk(0)
    for mh in range(2):
        pltpu.async_copy(msg_v.at[mh], acc_sh.at[is_v.at[0, 0]],
                         ssems[mh], add=True)
    issue_gather(0, 0, 0)

    @pl.loop(0, n_super)
    def _super(sp):
        sb = sp & 1
        ngc_s = jnp.minimum(n_gch - sp * SUPER, SUPER)

        @pl.loop(0, (ngc_s + 1) // 2)
        def _pair(gp):
            t0 = 2 * gp

            @pl.when(t0 + 1 < ngc_s)
            def _():
                issue_gather(sb, t0 + 1, 1)

            wait_gather(sb, t0, 0)
            compute_sub(sb, 2 * t0, rows_bufs[0], 0, 0)
            compute_sub(sb, 2 * t0 + 1, rows_bufs[0], SCH, 1)

            @pl.when(t0 + 2 < ngc_s)
            def _():
                issue_gather(sb, t0 + 2, 0)

            @pl.when((gp == 0) & ((sp + 1) * SUPER < n_gch))
            def _():
                issue_pk(sp + 1, 1 - sb)

            @pl.when(t0 + 1 < ngc_s)
            def _():
                wait_gather(sb, t0 + 1, 1)
                compute_sub(sb, 2 * t0 + 2, rows_bufs[1], 0, 0)
                compute_sub(sb, 2 * t0 + 3, rows_bufs[1], SCH, 1)

        @pl.when((sp + 1) * SUPER < n_gch)
        def _():
            wait_pk(1 - sb)
            issue_gather(1 - sb, 0, 0)

    wait_scatter(0)
    wait_scatter(1)
    plsc.subcore_barrier()
    pltpu.sync_copy(acc_sh.at[pl.ds(s * ROWS_PER_TILE, ROWS_PER_TILE)],
                    part_hbm.at[c, pl.ds(s * ROWS_PER_TILE, ROWS_PER_TILE)])


_sc_call = pl.kernel(
    _sc_body,
    out_type=jax.ShapeDtypeStruct((NC, NPAD, F), jnp.float32),
    mesh=plsc.VectorSubcoreMesh(core_axis_name="c", subcore_axis_name="s"),
    compiler_params=pltpu.CompilerParams(use_tc_tiling_on_sc=False,
                                         needs_layout_passes=False),
    scratch_types=[
        pltpu.VMEM((2, SUPER, GCH), jnp.int32),
        pltpu.VMEM((2, 2 * SUPER, SCH), jnp.int32),
        pltpu.VMEM((2, 2 * SUPER, SCH), jnp.int32),
        pltpu.VMEM((2, GCH, YW), jnp.bfloat16),
        pltpu.VMEM((2, SCH, F), jnp.float32),
        pltpu.VMEM_SHARED((NPAD, F), jnp.float32),
        pltpu.SemaphoreType.DMA,
        pltpu.SemaphoreType.DMA,
        pltpu.SemaphoreType.DMA,
        pltpu.SemaphoreType.DMA,
        pltpu.SemaphoreType.DMA,
    ],
)


@jax.jit
def kernel(x, edge_index, edge_attr, weight):
    w_flat = weight.transpose(2, 0, 1, 3).reshape(F, YW)[:, _PERM]
    grid = N_NODES // _MM_BLOCK
    y = pl.pallas_call(
        _mm_body,
        grid=(grid,),
        in_specs=[
            pl.BlockSpec((_MM_BLOCK, F), lambda i: (i, 0)),
            pl.BlockSpec((F, YW), lambda i: (0, 0)),
        ],
        out_specs=pl.BlockSpec((_MM_BLOCK, YW), lambda i: (i, 0)),
        out_shape=jax.ShapeDtypeStruct((N_NODES, YW), jnp.bfloat16),
    )(x, w_flat)

    jg = edge_index[1].reshape(NGCH, GCH)
    i2 = edge_index[0].reshape(NSCH, SCH)
    pq = lax.bitcast_convert_type(edge_attr.astype(jnp.bfloat16),
                                  jnp.int32).reshape(NSCH, SCH)

    partials = _sc_call(y, jg, i2, pq)

    out = pl.pallas_call(
        _add_body,
        grid=(grid,),
        in_specs=[pl.BlockSpec((NC, _MM_BLOCK, F), lambda i: (0, i, 0))],
        out_specs=pl.BlockSpec((_MM_BLOCK, F), lambda i: (i, 0)),
        out_shape=jax.ShapeDtypeStruct((N_NODES, F), jnp.float32),
    )(partials)
    return out
